# Initial kernel scaffold; baseline (speedup 1.0000x reference)
import jax
import jax.numpy as jnp
from jax.experimental import pallas as pl


def kernel(x, norm_w, W_router, U, G, V):
    raise NotImplementedError("write your pallas kernel here")


# speedup vs baseline: 3.1218x; 3.1218x over previous
import functools

import jax
import jax.numpy as jnp
from jax.experimental import pallas as pl

D = 2048
E = 8
K = 2
R3 = 512
R2 = 512
B = 4096
EPS = 1e-5
SCALE = 10.0
TEMP = 0.5

T = 512


def _moe_body(x_ref, nw_ref, wr_ref, u_ref, g_ref, v_ref, o_ref):
    x = x_ref[...]
    var = jnp.mean(x * x, axis=-1, keepdims=True)
    xn = x * jax.lax.rsqrt(var + EPS) * nw_ref[...]

    logits = jnp.dot(xn, wr_ref[...], preferred_element_type=jnp.float32)
    col = jax.lax.broadcasted_iota(jnp.int32, (T, E), 1)
    m1 = jnp.max(logits, axis=-1, keepdims=True)
    i1 = jnp.min(jnp.where(logits == m1, col, E), axis=-1, keepdims=True)
    masked = jnp.where(col == i1, -jnp.inf, logits)
    m2 = jnp.max(masked, axis=-1, keepdims=True)
    i2 = jnp.min(jnp.where(masked == m2, col, E), axis=-1, keepdims=True)
    b = jnp.exp((m2 - m1) / TEMP)
    p1 = 1.0 / (1.0 + b)
    p2 = 1.0 - p1
    w = jnp.where(col == i1, p1, 0.0) + jnp.where(col == i2, p2, 0.0)

    xs = jnp.tanh(jnp.dot(xn, u_ref[...], preferred_element_type=jnp.float32)
                  * (1.0 / SCALE)) * SCALE

    acc = jnp.zeros((T, R2), dtype=jnp.float32)
    for e in range(E):
        he = jnp.dot(xs, g_ref[e], preferred_element_type=jnp.float32)
        acc = acc + w[:, e:e + 1] * he

    o_ref[...] = jnp.dot(acc, v_ref[...], preferred_element_type=jnp.float32)


@jax.jit
def kernel(x, norm_w, W_router, U, G, V):
    grid = (B // T,)
    return pl.pallas_call(
        _moe_body,
        grid=grid,
        in_specs=[
            pl.BlockSpec((T, D), lambda i: (i, 0)),
            pl.BlockSpec((1, D), lambda i: (0, 0)),
            pl.BlockSpec((D, E), lambda i: (0, 0)),
            pl.BlockSpec((D, R3), lambda i: (0, 0)),
            pl.BlockSpec((E, R3, R2), lambda i: (0, 0, 0)),
            pl.BlockSpec((R2, D), lambda i: (0, 0)),
        ],
        out_specs=pl.BlockSpec((T, D), lambda i: (i, 0)),
        out_shape=jax.ShapeDtypeStruct((B, D), jnp.float32),
    )(x, norm_w.reshape(1, D), W_router, U, G, V)
